# Initial kernel scaffold; baseline (speedup 1.0000x reference)
import jax
import jax.numpy as jnp
from jax.experimental import pallas as pl


def kernel(inputs, embedding):
    raise NotImplementedError("write your pallas kernel here")


# speedup vs baseline: 1.8043x; 1.8043x over previous
import jax
import jax.numpy as jnp
from jax.experimental import pallas as pl
from jax.experimental.pallas import tpu as pltpu

_K = 1024
_D = 64
_B = 2048
_COMMITMENT_COST = 1.0


def _vq_block(x_ref, e_ref, q_ref, idx_ref, loss_ref):
    i = pl.program_id(0)
    x = x_ref[:]
    e = e_ref[:]
    xn = jnp.sum(x * x, axis=1, keepdims=True)
    en = jnp.sum(e * e, axis=1)
    prod = jax.lax.dot_general(
        x, e, (((1,), (1,)), ((), ())), preferred_element_type=jnp.float32
    )
    dist = xn + en[None, :] - 2.0 * prod
    idx = jnp.argmin(dist, axis=1).astype(jnp.int32)
    idx_ref[:] = idx
    iota = jax.lax.broadcasted_iota(jnp.int32, (_B, _K), 1)
    oh = (idx[:, None] == iota).astype(jnp.float32)
    q = jax.lax.dot_general(
        oh, e, (((1,), (0,)), ((), ())), preferred_element_type=jnp.float32
    )
    q_ref[:] = q

    part = jnp.sum((x - q) ** 2)

    @pl.when(i == 0)
    def _init():
        loss_ref[0, 0] = 0.0

    loss_ref[0, 0] += part

    @pl.when(i == pl.num_programs(0) - 1)
    def _fini():
        loss_ref[0, 0] = loss_ref[0, 0] / (pl.num_programs(0) * _B * _D)


def kernel(inputs, embedding):
    x = inputs.reshape(-1, _D)
    n = x.shape[0]
    q, idx, loss = pl.pallas_call(
        _vq_block,
        grid=(n // _B,),
        in_specs=[
            pl.BlockSpec((_B, _D), lambda i: (i, 0)),
            pl.BlockSpec((_K, _D), lambda i: (0, 0)),
        ],
        out_specs=[
            pl.BlockSpec((_B, _D), lambda i: (i, 0)),
            pl.BlockSpec((_B,), lambda i: (i,)),
            pl.BlockSpec((1, 1), lambda i: (0, 0), memory_space=pltpu.SMEM),
        ],
        out_shape=[
            jax.ShapeDtypeStruct((n, _D), jnp.float32),
            jax.ShapeDtypeStruct((n,), jnp.int32),
            jax.ShapeDtypeStruct((1, 1), jnp.float32),
        ],
    )(x, embedding)
    return q, _COMMITMENT_COST * loss[0, 0], idx
